# baseline (speedup 1.0000x reference)
import functools

import jax
import jax.numpy as jnp
from jax import lax
from jax.experimental import pallas as pl
from jax.experimental.pallas import tpu as pltpu
from jax.experimental.pallas import tpu_sc as plsc

_B, _C, _N, _K, _OUT = 2, 64, 4096, 20, 64
_BM = 1024
_KPAD = 24
_BN = _B * _N
_NW = 32
_PPW = _BN // _NW
_P = 32
_NCHUNK = _PPW // _P
_IPG = 128
_GPC = _P * _K // _IPG


def _knn_topk_kernel(xt_ref, x_ref, w1_ref, wd_ref, idx_ref, p_ref, q_ref,
                     d_ref):
    xt = xt_ref[0]
    xfull = x_ref[0]
    p_ref[0] = jnp.dot(xt, w1_ref[...], preferred_element_type=jnp.float32)
    q_ref[0] = jnp.dot(xt, wd_ref[...], preferred_element_type=jnp.float32)
    dot = jnp.dot(xt, xfull, preferred_element_type=jnp.float32)
    xx = jnp.sum(xfull * xfull, axis=0)[None, :]
    d_ref[...] = 2.0 * dot - xx
    neg = jnp.float32(-jnp.inf)

    def group(i, _):
        dr = d_ref[pl.ds(i * 8, 8), :]
        lanes = jax.lax.broadcasted_iota(jnp.int32, dr.shape, 1)
        for j in range(_K):
            am = jnp.argmax(dr, axis=1).astype(jnp.int32)
            idx_ref[0, pl.ds(i * 8, 8), j] = am
            dr = jnp.where(lanes == am[:, None], neg, dr)
        return 0

    jax.lax.fori_loop(0, _BM // 8, group, 0)
    idx_ref[0, :, _K:_KPAD] = jnp.zeros((_BM, _KPAD - _K), jnp.int32)


def _knn_topk(x, xt, w1t, wdt):
    grid = (_B, _N // _BM)
    idx, p_t, q_t = pl.pallas_call(
        _knn_topk_kernel,
        grid=grid,
        in_specs=[
            pl.BlockSpec((1, _BM, _C), lambda b, i: (b, i, 0)),
            pl.BlockSpec((1, _C, _N), lambda b, i: (b, 0, 0)),
            pl.BlockSpec((_C, _OUT), lambda b, i: (0, 0)),
            pl.BlockSpec((_C, _OUT), lambda b, i: (0, 0)),
        ],
        out_specs=[
            pl.BlockSpec((1, _BM, _KPAD), lambda b, i: (b, i, 0)),
            pl.BlockSpec((1, _BM, _OUT), lambda b, i: (b, i, 0)),
            pl.BlockSpec((1, _BM, _OUT), lambda b, i: (b, i, 0)),
        ],
        out_shape=[
            jax.ShapeDtypeStruct((_B, _N, _KPAD), jnp.int32),
            jax.ShapeDtypeStruct((_B, _N, _OUT), jnp.float32),
            jax.ShapeDtypeStruct((_B, _N, _OUT), jnp.float32),
        ],
        scratch_shapes=[pltpu.VMEM((_BM, _N), jnp.float32)],
        compiler_params=pltpu.CompilerParams(
            dimension_semantics=("parallel", "arbitrary")),
    )(xt, x, w1t, wdt)
    return idx[:, :, :_K], p_t, q_t


def _sc_gather_reduce(p_rows, idx2d):
    mesh = plsc.VectorSubcoreMesh(core_axis_name="c", subcore_axis_name="s")
    sds = jax.ShapeDtypeStruct((_BN, _OUT), jnp.float32)

    @functools.partial(
        pl.kernel, mesh=mesh,
        out_type=(sds, sds, sds),
        compiler_params=pltpu.CompilerParams(use_tc_tiling_on_sc=False),
        scratch_types=[
            pltpu.VMEM((_PPW * _K // _IPG, _IPG), jnp.int32),
            pltpu.VMEM((_P * _K, _OUT), jnp.float32),
            pltpu.VMEM((_P, _OUT), jnp.float32),
            pltpu.VMEM((_P, _OUT), jnp.float32),
            pltpu.VMEM((_P, _OUT), jnp.float32),
            pltpu.SemaphoreType.DMA,
        ],
    )
    def k(p_hbm, i_hbm, mx_hbm, s1_hbm, s2_hbm,
          idx_v, gath_v, mx_v, s1_v, s2_v, sem):
        wid = lax.axis_index("s") * 2 + lax.axis_index("c")
        pt0 = wid * _PPW
        irow0 = pl.multiple_of(pt0 * _K // _IPG, 8)
        pltpu.sync_copy(i_hbm.at[pl.ds(irow0, _PPW * _K // _IPG)], idx_v)

        @pl.loop(0, _NCHUNK)
        def _chunk(ci):
            pbase = pt0 + ci * _P
            copies = []
            for g in range(_GPC):
                copies.append(pltpu.async_copy(
                    p_hbm.at[idx_v.at[ci * _GPC + g]],
                    gath_v.at[pl.ds(g * _IPG, _IPG)], sem))
            for cp in copies:
                cp.wait()

            @pl.loop(0, _P)
            def _point(g):
                rb = g * _K
                for c in range(_OUT // 16):
                    sl = pl.ds(c * 16, 16)
                    v = gath_v.at[rb, sl][...]
                    m = v
                    s = v
                    ss = v * v
                    for j in range(1, _K):
                        v = gath_v.at[rb + j, sl][...]
                        m = jnp.maximum(m, v)
                        s = s + v
                        ss = ss + v * v
                    mx_v.at[g, sl][...] = m
                    s1_v.at[g, sl][...] = s
                    s2_v.at[g, sl][...] = ss

            pltpu.sync_copy(mx_v, mx_hbm.at[pl.ds(pbase, _P)])
            pltpu.sync_copy(s1_v, s1_hbm.at[pl.ds(pbase, _P)])
            pltpu.sync_copy(s2_v, s2_hbm.at[pl.ds(pbase, _P)])

    return k(p_rows, idx2d)


def _finalize_kernel(maxpq_ref, stats_ref, gamma_ref, beta_ref, out_ref):
    mean = stats_ref[0, :]
    var = stats_ref[1, :]
    inv = jax.lax.rsqrt(var + 1e-5) * gamma_ref[0, :]
    y = (maxpq_ref[...] - mean[None, :]) * inv[None, :] + beta_ref[0, :][None, :]
    out_ref[...] = jnp.where(y > 0, y, 0.2 * y)


def kernel(x, W, gamma, beta):
    B, C, N, K, OUT = _B, _C, _N, _K, _OUT
    W1, W2 = W[:, :C], W[:, C:]
    w1t = jnp.transpose(W1)
    wdt = jnp.transpose(W2 - W1)

    xt = jnp.transpose(x, (0, 2, 1))
    idx, p_t, q_t = _knn_topk(x, xt, w1t, wdt)

    idxg = idx + (jnp.arange(B) * N)[:, None, None]
    idx2d = idxg.reshape(_BN * K // _IPG, _IPG).astype(jnp.int32)
    p_rows = p_t.reshape(_BN, OUT)

    mx, s1, s2 = _sc_gather_reduce(p_rows, idx2d)

    q_rows = q_t.reshape(_BN, OUT)
    M = B * N * K
    S1 = jnp.sum(s1, axis=0) + K * jnp.sum(q_rows, axis=0)
    S2 = (jnp.sum(s2, axis=0) + 2.0 * jnp.sum(q_rows * s1, axis=0)
          + K * jnp.sum(q_rows * q_rows, axis=0))
    mean = S1 / M
    var = S2 / M - mean * mean
    stats = jnp.stack([mean, var], axis=0)

    out_rows = pl.pallas_call(
        _finalize_kernel,
        out_shape=jax.ShapeDtypeStruct((_BN, OUT), jnp.float32),
    )(mx + q_rows, stats, gamma[None, :], beta[None, :])
    return jnp.transpose(out_rows.reshape(B, N, OUT), (0, 2, 1))


# speedup vs baseline: 4.3879x; 4.3879x over previous
import functools

import jax
import jax.numpy as jnp
from jax import lax
from jax.experimental import pallas as pl
from jax.experimental.pallas import tpu as pltpu
from jax.experimental.pallas import tpu_sc as plsc

_B, _C, _N, _K, _OUT = 2, 64, 4096, 20, 64
_BM = 1024
_KPAD = 24
_BN = _B * _N
_NW = 32
_PPW = _BN // _NW
_P = 32
_NCHUNK = _PPW // _P
_IPG = 128
_GPC = _P * _K // _IPG


def _knn_topk_kernel(xt_ref, x_ref, w1_ref, wd_ref, idx_ref, p_ref, q_ref):
    xt = xt_ref[0]
    xfull = x_ref[0]
    p_ref[0] = jnp.dot(xt, w1_ref[...], preferred_element_type=jnp.float32)
    q_ref[0] = jnp.dot(xt, wd_ref[...], preferred_element_type=jnp.float32)
    dot = jnp.dot(xt, xfull, preferred_element_type=jnp.float32)
    xx = jnp.sum(xfull * xfull, axis=0)[None, :]
    d = 2.0 * dot - xx
    lanes = jax.lax.broadcasted_iota(jnp.int32, d.shape, 1)
    neg = jnp.float32(-jnp.inf)
    for j in range(_K):
        am = jnp.argmax(d, axis=1).astype(jnp.int32)
        idx_ref[0, j, :] = am
        d = jnp.where(lanes == am[:, None], neg, d)
    for j in range(_K, _KPAD):
        idx_ref[0, j, :] = jnp.zeros((_BM,), jnp.int32)


def _knn_topk(x, xt, w1t, wdt):
    grid = (_B, _N // _BM)
    idx, p_t, q_t = pl.pallas_call(
        _knn_topk_kernel,
        grid=grid,
        in_specs=[
            pl.BlockSpec((1, _BM, _C), lambda b, i: (b, i, 0)),
            pl.BlockSpec((1, _C, _N), lambda b, i: (b, 0, 0)),
            pl.BlockSpec((_C, _OUT), lambda b, i: (0, 0)),
            pl.BlockSpec((_C, _OUT), lambda b, i: (0, 0)),
        ],
        out_specs=[
            pl.BlockSpec((1, _KPAD, _BM), lambda b, i: (b, 0, i)),
            pl.BlockSpec((1, _BM, _OUT), lambda b, i: (b, i, 0)),
            pl.BlockSpec((1, _BM, _OUT), lambda b, i: (b, i, 0)),
        ],
        out_shape=[
            jax.ShapeDtypeStruct((_B, _KPAD, _N), jnp.int32),
            jax.ShapeDtypeStruct((_B, _N, _OUT), jnp.float32),
            jax.ShapeDtypeStruct((_B, _N, _OUT), jnp.float32),
        ],
        compiler_params=pltpu.CompilerParams(
            dimension_semantics=("parallel", "arbitrary")),
    )(xt, x, w1t, wdt)
    return idx[:, :_K, :], p_t, q_t


def _sc_gather_reduce(p_rows, idx2d):
    mesh = plsc.VectorSubcoreMesh(core_axis_name="c", subcore_axis_name="s")
    sds = jax.ShapeDtypeStruct((_BN, _OUT), jnp.float32)

    @functools.partial(
        pl.kernel, mesh=mesh,
        out_type=(sds, sds, sds),
        compiler_params=pltpu.CompilerParams(use_tc_tiling_on_sc=False),
        scratch_types=[
            pltpu.VMEM((_PPW * _K // _IPG, _IPG), jnp.int32),
            pltpu.VMEM((_P * _K, _OUT), jnp.float32),
            pltpu.VMEM((_P * _K, _OUT), jnp.float32),
            pltpu.VMEM((_P, _OUT), jnp.float32),
            pltpu.VMEM((_P, _OUT), jnp.float32),
            pltpu.VMEM((_P, _OUT), jnp.float32),
            pltpu.SemaphoreType.DMA,
            pltpu.SemaphoreType.DMA,
        ],
    )
    def k(p_hbm, i_hbm, mx_hbm, s1_hbm, s2_hbm,
          idx_v, gath_a, gath_b, mx_v, s1_v, s2_v, sem_a, sem_b):
        wid = lax.axis_index("s") * 2 + lax.axis_index("c")
        pt0 = wid * _PPW
        irow0 = pl.multiple_of(pt0 * _K // _IPG, 8)
        pltpu.sync_copy(i_hbm.at[pl.ds(irow0, _PPW * _K // _IPG)], idx_v)

        def fire(ci, buf, sem):
            for g in range(_GPC):
                pltpu.make_async_copy(
                    p_hbm.at[idx_v.at[ci * _GPC + g]],
                    buf.at[pl.ds(g * _IPG, _IPG)], sem).start()

        def drain(ci, buf, sem):
            for g in range(_GPC):
                pltpu.make_async_copy(
                    p_hbm.at[idx_v.at[ci * _GPC + g]],
                    buf.at[pl.ds(g * _IPG, _IPG)], sem).wait()

        def compute(ci, buf):
            pbase = pt0 + ci * _P

            @pl.loop(0, _P)
            def _point(g):
                rb = g * _K
                for c in range(_OUT // 16):
                    sl = pl.ds(c * 16, 16)
                    v = buf.at[rb, sl][...]
                    m = v
                    s = v
                    ss = v * v
                    for j in range(1, _K):
                        v = buf.at[rb + j, sl][...]
                        m = jnp.maximum(m, v)
                        s = s + v
                        ss = ss + v * v
                    mx_v.at[g, sl][...] = m
                    s1_v.at[g, sl][...] = s
                    s2_v.at[g, sl][...] = ss

            pltpu.sync_copy(mx_v, mx_hbm.at[pl.ds(pbase, _P)])
            pltpu.sync_copy(s1_v, s1_hbm.at[pl.ds(pbase, _P)])
            pltpu.sync_copy(s2_v, s2_hbm.at[pl.ds(pbase, _P)])

        fire(0, gath_a, sem_a)

        @pl.loop(0, _NCHUNK, step=2)
        def _chunk(ci):
            fire(ci + 1, gath_b, sem_b)
            drain(ci, gath_a, sem_a)
            compute(ci, gath_a)

            @pl.when(ci + 2 < _NCHUNK)
            def _():
                fire(ci + 2, gath_a, sem_a)

            drain(ci + 1, gath_b, sem_b)
            compute(ci + 1, gath_b)

    return k(p_rows, idx2d)


def _finalize_kernel(maxpq_ref, stats_ref, gamma_ref, beta_ref, out_ref):
    mean = stats_ref[0, :]
    var = stats_ref[1, :]
    inv = jax.lax.rsqrt(var + 1e-5) * gamma_ref[0, :]
    y = (maxpq_ref[...] - mean[None, :]) * inv[None, :] + beta_ref[0, :][None, :]
    out_ref[...] = jnp.where(y > 0, y, 0.2 * y)


def kernel(x, W, gamma, beta):
    B, C, N, K, OUT = _B, _C, _N, _K, _OUT
    W1, W2 = W[:, :C], W[:, C:]
    w1t = jnp.transpose(W1)
    wdt = jnp.transpose(W2 - W1)

    xt = jnp.transpose(x, (0, 2, 1))
    idx, p_t, q_t = _knn_topk(x, xt, w1t, wdt)

    idxg = jnp.transpose(idx, (0, 2, 1)) + (jnp.arange(B) * N)[:, None, None]
    idx2d = idxg.reshape(_BN * K // _IPG, _IPG).astype(jnp.int32)
    p_rows = p_t.reshape(_BN, OUT)

    mx, s1, s2 = _sc_gather_reduce(p_rows, idx2d)

    q_rows = q_t.reshape(_BN, OUT)
    M = B * N * K
    S1 = jnp.sum(s1, axis=0) + K * jnp.sum(q_rows, axis=0)
    S2 = (jnp.sum(s2, axis=0) + 2.0 * jnp.sum(q_rows * s1, axis=0)
          + K * jnp.sum(q_rows * q_rows, axis=0))
    mean = S1 / M
    var = S2 / M - mean * mean
    stats = jnp.stack([mean, var], axis=0)

    out_rows = pl.pallas_call(
        _finalize_kernel,
        out_shape=jax.ShapeDtypeStruct((_BN, OUT), jnp.float32),
    )(mx + q_rows, stats, gamma[None, :], beta[None, :])
    return jnp.transpose(out_rows.reshape(B, N, OUT), (0, 2, 1))
